# baseline (speedup 1.0000x reference)
import functools
import jax
import jax.numpy as jnp
from jax import lax
from jax.experimental import pallas as pl
from jax.experimental.pallas import tpu as pltpu

N_CIR = 585
N_DIS = 88
H = 8
C = 128


def _dot_nt(a, b):
    return lax.dot_general(a, b, (((1,), (1,)), ((), ())),
                           preferred_element_type=jnp.float32)


def _dot_nn(a, b):
    return lax.dot_general(a, b, (((1,), (0,)), ((), ())),
                           preferred_element_type=jnp.float32)


def _hist_t(edges_ref, cnt_ref, n, nb):
    e = edges_ref.shape[1]
    chunk = e // nb
    r_all = edges_ref[0:1, :]
    c_all = edges_ref[1:2, :]
    for i in range(nb):
        rows = r_all[:, i * chunk:(i + 1) * chunk]
        cols = c_all[:, i * chunk:(i + 1) * chunk]
        iota = lax.broadcasted_iota(jnp.int32, (n, chunk), 0)
        rt = jnp.where(iota == jnp.broadcast_to(rows, (n, chunk)),
                       1.0, 0.0).astype(jnp.bfloat16)
        ct = jnp.where(iota == jnp.broadcast_to(cols, (n, chunk)),
                       1.0, 0.0).astype(jnp.bfloat16)
        acc = _dot_nt(ct, rt)
        if i == 0:
            cnt_ref[...] = acc
        else:
            cnt_ref[...] += acc


def _gcn(st, xw, dinv, b):
    out = _dot_nn(st, xw * dinv) * dinv
    return jax.nn.relu(out + dinv * dinv * xw + b)


def _gat(cnt_t, mat_t, st, x1, wg, asrc, adst, we, ae, einv, bg):
    m = x1.shape[0]
    xs = _dot_nt(x1, wg)
    mean_ea = jnp.sum(st, axis=1, keepdims=True).sum(axis=0, keepdims=True) * einv
    present = cnt_t > 0.0
    acc = jnp.zeros((m, C), jnp.float32)
    for h in range(H):
        xs_h = xs[:, h * C:(h + 1) * C]
        asrc_h = asrc[h:h + 1, :]
        adst_h = adst[h:h + 1, :]
        coeff_h = _dot_nn(ae[h:h + 1, :], we[h * C:(h + 1) * C, :])
        a_src_col = jnp.sum(xs_h * asrc_h, axis=1, keepdims=True)
        a_dst_col = jnp.sum(xs_h * adst_h, axis=1, keepdims=True)
        a_src_row = _dot_nt(asrc_h, xs_h)
        alpha = a_dst_col + a_src_row + mat_t * coeff_h
        alpha = jnp.where(alpha > 0, alpha, 0.2 * alpha)
        aloop = a_src_col + a_dst_col + mean_ea * coeff_h
        aloop = jnp.where(aloop > 0, aloop, 0.2 * aloop)
        amax = jnp.max(jnp.where(present, alpha, -1e30), axis=1, keepdims=True)
        amax = jnp.maximum(amax, aloop)
        ex = cnt_t * jnp.exp(jnp.where(present, alpha - amax, -30.0))
        exl = jnp.exp(aloop - amax)
        den = jnp.sum(ex, axis=1, keepdims=True) + exl
        num = _dot_nn(ex, xs_h) + exl * xs_h
        acc = acc + num / (den + 1e-16)
    return jax.nn.relu(acc * (1.0 / H) + bg)


def _branch(cnt_t, mat, x, w1, b1, wg, asrc, adst, we2, ae, einv, bg, w2, b2):
    mat_t = mat.T
    st = cnt_t * mat_t
    dinv = lax.rsqrt(jnp.sum(st, axis=1, keepdims=True) + 1.0)
    x1 = _gcn(st, _dot_nt(x, w1), dinv, b1)
    xa = _gat(cnt_t, mat_t, st, x1, wg, asrc, adst, we2, ae, einv, bg)
    x2 = _gcn(st, _dot_nt(xa, w2), dinv, b2)
    return x1, x2


def _body(e_cc, e_dd,
          edges_c, edges_d, mat_c, mat_d, x_c, x_d,
          w1c, b1c, wgc, asrc_c, adst_c, we2c, aec, bgc, w2c, b2c,
          w1d, b1d, wgd, asrc_d, adst_d, we2d, aed, bgd, w2d, b2d,
          wcc, bcc, wcd, bcd,
          score_ref, cir_ref, dis_ref, cntc_ref, cntd_ref):
    _hist_t(edges_c, cntc_ref, N_CIR, 8)
    _hist_t(edges_d, cntd_ref, N_DIS, 2)
    x1, x2 = _branch(cntc_ref[...], mat_c[...], x_c[...], w1c[...],
                     b1c[...][None, :],
                     wgc[...], asrc_c[...], adst_c[...], we2c[...], aec[...],
                     1.0 / e_cc, bgc[...][None, :], w2c[...],
                     b2c[...][None, :])
    y1, y2 = _branch(cntd_ref[...], mat_d[...], x_d[...], w1d[...],
                     b1d[...][None, :],
                     wgd[...], asrc_d[...], adst_d[...], we2d[...], aed[...],
                     1.0 / e_dd, bgd[...][None, :], w2d[...],
                     b2d[...][None, :])
    cir = _dot_nt(jnp.concatenate([x1, x2], axis=1), wcc[...]) + bcc[...][None, :]
    dis = _dot_nt(jnp.concatenate([y1, y2], axis=1), wcd[...]) + bcd[...][None, :]
    cir_ref[...] = cir
    dis_ref[...] = dis
    score_ref[...] = _dot_nt(cir, dis)


def kernel(cc_matrix, cc_edges, dd_matrix, dd_edges, x_cir, x_dis,
           W1c, b1c, Wgc, asrc_c, adst_c, We_c, ae_c, bg_c, W2c, b2c,
           W1d, b1d, Wgd, asrc_d, adst_d, We_d, ae_d, bg_d, W2d, b2d,
           Wcnn_c, bcnn_c, Wcnn_d, bcnn_d):
    e_cc = cc_edges.shape[1]
    e_dd = dd_edges.shape[1]

    out_shapes = (
        jax.ShapeDtypeStruct((N_CIR, N_DIS), jnp.float32),
        jax.ShapeDtypeStruct((N_CIR, 2 * C), jnp.float32),
        jax.ShapeDtypeStruct((N_DIS, 2 * C), jnp.float32),
    )
    return pl.pallas_call(
        functools.partial(_body, float(e_cc), float(e_dd)),
        out_shape=out_shapes,
        scratch_shapes=[pltpu.VMEM((N_CIR, N_CIR), jnp.float32),
                        pltpu.VMEM((N_DIS, N_DIS), jnp.float32)],
    )(cc_edges, dd_edges, cc_matrix, dd_matrix, x_cir, x_dis,
      W1c, b1c, Wgc, asrc_c, adst_c, We_c, ae_c,
      bg_c, W2c, b2c,
      W1d, b1d, Wgd, asrc_d, adst_d, We_d, ae_d,
      bg_d, W2d, b2d,
      Wcnn_c, bcnn_c, Wcnn_d, bcnn_d)


# speedup vs baseline: 1.0014x; 1.0014x over previous
import functools
import jax
import jax.numpy as jnp
from jax import lax
from jax.experimental import pallas as pl
from jax.experimental.pallas import tpu as pltpu
from jax.experimental.pallas import tpu_sc as plsc

N_CIR = 585
N_DIS = 88
H = 8
C = 128

_NC = 2
_NS = 16
_NW = _NC * _NS

_PR_C, _LD_C = 592, 640
_PR_D, _LD_D = 96, 128
_HW_C = _PR_C * _LD_C
_HW_D = _PR_D * _LD_D
_CHK_C = _HW_C // _NS
_CHK_D = _HW_D // _NS


def _stage_edges(rows_v, cols_v, idx_v, val_v, ebase, lo, hi, ld):
    n = rows_v.shape[0]
    nb, bw = idx_v.shape
    lanes = lax.iota(jnp.int32, 16)
    for i in range(n // 16):
        r = rows_v[pl.ds(i * 16, 16)]
        c = cols_v[pl.ds(i * 16, 16)]
        g = ebase + i * 16 + lanes
        ok = (g >= lo) & (g < hi)
        lin = c * ld + r
        j, k = (i * 16) // bw, (i * 16) % bw
        idx_v[j, pl.ds(k, 16)] = jnp.where(ok, lin, 0)
        val_v[j, pl.ds(k, 16)] = jnp.where(ok, 1.0, 0.0)
    for t in range(n, nb * bw, 16):
        j, k = t // bw, t % bw
        idx_v[j, pl.ds(k, 16)] = jnp.zeros((16,), jnp.int32)
        val_v[j, pl.ds(k, 16)] = jnp.zeros((16,), jnp.float32)


def _sc_hist(rows_cc, cols_cc, rows_dd, cols_dd):
    e_cc = rows_cc.shape[0]
    e_dd = rows_dd.shape[0]
    per_c = e_cc // _NW
    win_c = ((per_c + 7) // 8) * 8 + 8
    nb_c = (win_c + 127) // 128
    per_d = e_dd // _NW
    win_d = per_d
    nb_d = (win_d + 127) // 128

    mesh = plsc.VectorSubcoreMesh(core_axis_name="c", subcore_axis_name="s")

    @functools.partial(
        pl.kernel, mesh=mesh,
        out_type=(jax.ShapeDtypeStruct((_NC, _HW_C), jnp.float32),
                  jax.ShapeDtypeStruct((_NC, _HW_D), jnp.float32)),
        scratch_types=[
            pltpu.VMEM((win_c,), jnp.int32),
            pltpu.VMEM((win_c,), jnp.int32),
            pltpu.VMEM((nb_c, 128), jnp.int32),
            pltpu.VMEM((nb_c, 128), jnp.float32),
            pltpu.VMEM((win_d,), jnp.int32),
            pltpu.VMEM((win_d,), jnp.int32),
            pltpu.VMEM((nb_d, 128), jnp.int32),
            pltpu.VMEM((nb_d, 128), jnp.float32),
            pltpu.VMEM((_CHK_C,), jnp.float32),
            pltpu.VMEM_SHARED((_HW_C,), jnp.float32),
            pltpu.VMEM_SHARED((_HW_D,), jnp.float32),
        ],
    )
    def k(ecc_r, ecc_c, edd_r, edd_c, out_cc, out_dd, rbc, cbc, idxc, valc,
          rbd, cbd, idxd, vald, zbuf, sh_cc, sh_dd):
        cid = lax.axis_index("c")
        sid = lax.axis_index("s")
        wid = sid * _NC + cid

        def zstep(i, _):
            zbuf[pl.ds(i * 16, 16)] = jnp.zeros((16,), jnp.float32)
            return 0
        lax.fori_loop(0, _CHK_C // 16, zstep, 0)
        pltpu.sync_copy(zbuf, sh_cc.at[pl.ds(sid * _CHK_C, _CHK_C)])
        pltpu.sync_copy(zbuf.at[pl.ds(0, _CHK_D)],
                        sh_dd.at[pl.ds(sid * _CHK_D, _CHK_D)])

        lo_c = wid * per_c
        ebase_c = jnp.minimum(lo_c // 8 * 8, e_cc - win_c)
        pltpu.sync_copy(ecc_r.at[pl.ds(ebase_c, win_c)], rbc)
        pltpu.sync_copy(ecc_c.at[pl.ds(ebase_c, win_c)], cbc)
        _stage_edges(rbc, cbc, idxc, valc, ebase_c, lo_c, lo_c + per_c, _LD_C)

        lo_d = wid * per_d
        pltpu.sync_copy(edd_r.at[pl.ds(lo_d, win_d)], rbd)
        pltpu.sync_copy(edd_c.at[pl.ds(lo_d, win_d)], cbd)
        _stage_edges(rbd, cbd, idxd, vald, lo_d, lo_d, lo_d + per_d, _LD_D)

        plsc.subcore_barrier()

        for j in range(nb_c):
            pltpu.sync_copy(valc.at[j], sh_cc.at[idxc.at[j]], add=True)
        for j in range(nb_d):
            pltpu.sync_copy(vald.at[j], sh_dd.at[idxd.at[j]], add=True)

        plsc.subcore_barrier()

        pltpu.sync_copy(sh_cc.at[pl.ds(sid * _CHK_C, _CHK_C)],
                        out_cc.at[cid, pl.ds(sid * _CHK_C, _CHK_C)])
        pltpu.sync_copy(sh_dd.at[pl.ds(sid * _CHK_D, _CHK_D)],
                        out_dd.at[cid, pl.ds(sid * _CHK_D, _CHK_D)])

    return k(rows_cc, cols_cc, rows_dd, cols_dd)


def _dot_nt(a, b):
    return lax.dot_general(a, b, (((1,), (1,)), ((), ())),
                           preferred_element_type=jnp.float32)


def _dot_nn(a, b):
    return lax.dot_general(a, b, (((1,), (0,)), ((), ())),
                           preferred_element_type=jnp.float32)


def _hist_t(edges_ref, cnt_ref, n, nb):
    e = edges_ref.shape[1]
    chunk = e // nb
    r_all = edges_ref[0:1, :]
    c_all = edges_ref[1:2, :]
    for i in range(nb):
        rows = r_all[:, i * chunk:(i + 1) * chunk]
        cols = c_all[:, i * chunk:(i + 1) * chunk]
        iota = lax.broadcasted_iota(jnp.int32, (n, chunk), 0)
        rt = jnp.where(iota == jnp.broadcast_to(rows, (n, chunk)),
                       1.0, 0.0).astype(jnp.bfloat16)
        ct = jnp.where(iota == jnp.broadcast_to(cols, (n, chunk)),
                       1.0, 0.0).astype(jnp.bfloat16)
        acc = _dot_nt(ct, rt)
        if i == 0:
            cnt_ref[...] = acc
        else:
            cnt_ref[...] += acc


def _gcn(st, xw, dinv, b):
    out = _dot_nn(st, xw * dinv) * dinv
    return jax.nn.relu(out + dinv * dinv * xw + b)


def _gat(cnt_t, mat_t, st, x1, wg, asrc, adst, we, ae, einv, bg):
    m = x1.shape[0]
    xs = _dot_nt(x1, wg)
    mean_ea = jnp.sum(st, axis=1, keepdims=True).sum(axis=0, keepdims=True) * einv
    present = cnt_t > 0.0
    acc = jnp.zeros((m, C), jnp.float32)
    for h in range(H):
        xs_h = xs[:, h * C:(h + 1) * C]
        asrc_h = asrc[h:h + 1, :]
        adst_h = adst[h:h + 1, :]
        coeff_h = jnp.sum(we[h:h + 1, :] * ae[h:h + 1, :], axis=1,
                          keepdims=True)
        a_src_col = jnp.sum(xs_h * asrc_h, axis=1, keepdims=True)
        a_dst_col = jnp.sum(xs_h * adst_h, axis=1, keepdims=True)
        a_src_row = _dot_nt(asrc_h, xs_h)
        alpha = a_dst_col + a_src_row + mat_t * coeff_h
        alpha = jnp.where(alpha > 0, alpha, 0.2 * alpha)
        aloop = a_src_col + a_dst_col + mean_ea * coeff_h
        aloop = jnp.where(aloop > 0, aloop, 0.2 * aloop)
        amax = jnp.max(jnp.where(present, alpha, -1e30), axis=1, keepdims=True)
        amax = jnp.maximum(amax, aloop)
        ex = cnt_t * jnp.exp(jnp.where(present, alpha - amax, -30.0))
        exl = jnp.exp(aloop - amax)
        den = jnp.sum(ex, axis=1, keepdims=True) + exl
        num = _dot_nn(ex, xs_h) + exl * xs_h
        acc = acc + num / (den + 1e-16)
    return jax.nn.relu(acc * (1.0 / H) + bg)


def _branch(cnt_t, mat, x, w1, b1, wg, asrc, adst, we2, ae, einv, bg, w2, b2):
    mat_t = mat.T
    st = cnt_t * mat_t
    dinv = lax.rsqrt(jnp.sum(st, axis=1, keepdims=True) + 1.0)
    x1 = _gcn(st, _dot_nt(x, w1), dinv, b1)
    xa = _gat(cnt_t, mat_t, st, x1, wg, asrc, adst, we2, ae, einv, bg)
    x2 = _gcn(st, _dot_nt(xa, w2), dinv, b2)
    return x1, x2


def _body(e_cc, e_dd,
          hc_ref, hd_ref, mat_c, mat_d, x_c, x_d,
          w1c, b1c, wgc, asrc_c, adst_c, we2c, aec, bgc, w2c, b2c,
          w1d, b1d, wgd, asrc_d, adst_d, we2d, aed, bgd, w2d, b2d,
          wcc, bcc, wcd, bcd,
          score_ref, cir_ref, dis_ref):
    hc = hc_ref[...]
    hd = hd_ref[...]
    cnt_c = (hc[0] + hc[1])[:N_CIR, :N_CIR]
    cnt_d = (hd[0] + hd[1])[:N_DIS, :N_DIS]
    x1, x2 = _branch(cnt_c, mat_c[...], x_c[...], w1c[...],
                     b1c[...][None, :],
                     wgc[...], asrc_c[...], adst_c[...], we2c[...], aec[...],
                     1.0 / e_cc, bgc[...][None, :], w2c[...],
                     b2c[...][None, :])
    y1, y2 = _branch(cnt_d, mat_d[...], x_d[...], w1d[...],
                     b1d[...][None, :],
                     wgd[...], asrc_d[...], adst_d[...], we2d[...], aed[...],
                     1.0 / e_dd, bgd[...][None, :], w2d[...],
                     b2d[...][None, :])
    cir = _dot_nt(jnp.concatenate([x1, x2], axis=1), wcc[...]) + bcc[...][None, :]
    dis = _dot_nt(jnp.concatenate([y1, y2], axis=1), wcd[...]) + bcd[...][None, :]
    cir_ref[...] = cir
    dis_ref[...] = dis
    score_ref[...] = _dot_nt(cir, dis)


def kernel(cc_matrix, cc_edges, dd_matrix, dd_edges, x_cir, x_dis,
           W1c, b1c, Wgc, asrc_c, adst_c, We_c, ae_c, bg_c, W2c, b2c,
           W1d, b1d, Wgd, asrc_d, adst_d, We_d, ae_d, bg_d, W2d, b2d,
           Wcnn_c, bcnn_c, Wcnn_d, bcnn_d):
    e_cc = cc_edges.shape[1]
    e_dd = dd_edges.shape[1]

    hist_cc, hist_dd = _sc_hist(cc_edges[0], cc_edges[1],
                                dd_edges[0], dd_edges[1])
    hist_cc = hist_cc.reshape(_NC, _PR_C, _LD_C)
    hist_dd = hist_dd.reshape(_NC, _PR_D, _LD_D)

    out_shapes = (
        jax.ShapeDtypeStruct((N_CIR, N_DIS), jnp.float32),
        jax.ShapeDtypeStruct((N_CIR, 2 * C), jnp.float32),
        jax.ShapeDtypeStruct((N_DIS, 2 * C), jnp.float32),
    )
    return pl.pallas_call(
        functools.partial(_body, float(e_cc), float(e_dd)),
        out_shape=out_shapes,
    )(hist_cc, hist_dd, cc_matrix, dd_matrix, x_cir, x_dis,
      W1c, b1c, Wgc, asrc_c, adst_c, We_c.reshape(8, 128), ae_c,
      bg_c, W2c, b2c,
      W1d, b1d, Wgd, asrc_d, adst_d, We_d.reshape(8, 128), ae_d,
      bg_d, W2d, b2d,
      Wcnn_c, bcnn_c, Wcnn_d, bcnn_d)


# speedup vs baseline: 1.1222x; 1.1207x over previous
import functools
import jax
import jax.numpy as jnp
from jax import lax
from jax.experimental import pallas as pl
from jax.experimental.pallas import tpu as pltpu
from jax.experimental.pallas import tpu_sc as plsc

N_CIR = 585
N_DIS = 88
H = 8
C = 128

_NC = 2
_NS = 16
_NW = _NC * _NS

_PR_C, _LD_C = 592, 640
_PR_D, _LD_D = 96, 128
_HW_C = _PR_C * _LD_C
_HW_D = _PR_D * _LD_D
_CHK_C = _HW_C // _NS
_CHK_D = _HW_D // _NS


def _stage_edges(rows_v, cols_v, idx_v, val_v, ebase, lo, hi, ld):
    n = rows_v.shape[0]
    nb, bw = idx_v.shape
    lanes = lax.iota(jnp.int32, 16)
    for i in range(n // 16):
        r = rows_v[pl.ds(i * 16, 16)]
        c = cols_v[pl.ds(i * 16, 16)]
        g = ebase + i * 16 + lanes
        ok = (g >= lo) & (g < hi)
        lin = c * ld + r
        j, k = (i * 16) // bw, (i * 16) % bw
        idx_v[j, pl.ds(k, 16)] = jnp.where(ok, lin, 0)
        val_v[j, pl.ds(k, 16)] = jnp.where(ok, 1.0, 0.0)
    for t in range(n, nb * bw, 16):
        j, k = t // bw, t % bw
        idx_v[j, pl.ds(k, 16)] = jnp.zeros((16,), jnp.int32)
        val_v[j, pl.ds(k, 16)] = jnp.zeros((16,), jnp.float32)


def _sc_hist(rows_cc, cols_cc, rows_dd, cols_dd):
    e_cc = rows_cc.shape[0]
    e_dd = rows_dd.shape[0]
    per_c = e_cc // _NW
    win_c = ((per_c + 7) // 8) * 8 + 8
    nb_c = (win_c + 127) // 128
    per_d = e_dd // _NW
    win_d = per_d
    nb_d = (win_d + 127) // 128

    mesh = plsc.VectorSubcoreMesh(core_axis_name="c", subcore_axis_name="s")

    @functools.partial(
        pl.kernel, mesh=mesh,
        out_type=(jax.ShapeDtypeStruct((_NC, _HW_C), jnp.float32),
                  jax.ShapeDtypeStruct((_NC, _HW_D), jnp.float32)),
        scratch_types=[
            pltpu.VMEM((win_c,), jnp.int32),
            pltpu.VMEM((win_c,), jnp.int32),
            pltpu.VMEM((nb_c, 128), jnp.int32),
            pltpu.VMEM((nb_c, 128), jnp.float32),
            pltpu.VMEM((win_d,), jnp.int32),
            pltpu.VMEM((win_d,), jnp.int32),
            pltpu.VMEM((nb_d, 128), jnp.int32),
            pltpu.VMEM((nb_d, 128), jnp.float32),
            pltpu.VMEM((_CHK_C // 10,), jnp.float32),
            pltpu.VMEM_SHARED((_HW_C,), jnp.float32),
            pltpu.VMEM_SHARED((_HW_D,), jnp.float32),
            pltpu.SemaphoreType.DMA,
        ],
    )
    def k(ecc_r, ecc_c, edd_r, edd_c, out_cc, out_dd, rbc, cbc, idxc, valc,
          rbd, cbd, idxd, vald, zbuf, sh_cc, sh_dd, sem):
        cid = lax.axis_index("c")
        sid = lax.axis_index("s")
        wid = sid * _NC + cid

        zc = _CHK_C // 10
        def zstep(i, _):
            zbuf[pl.ds(i * 16, 16)] = jnp.zeros((16,), jnp.float32)
            return 0
        lax.fori_loop(0, zc // 16, zstep, 0)
        zcopies = [pltpu.async_copy(
            zbuf, sh_cc.at[pl.ds(sid * _CHK_C + j * zc, zc)], sem)
            for j in range(10)]
        zcopies.append(pltpu.async_copy(
            zbuf.at[pl.ds(0, _CHK_D)],
            sh_dd.at[pl.ds(sid * _CHK_D, _CHK_D)], sem))
        for cp in zcopies:
            cp.wait()

        lo_c = wid * per_c
        ebase_c = jnp.minimum(lo_c // 8 * 8, e_cc - win_c)
        pltpu.sync_copy(ecc_r.at[pl.ds(ebase_c, win_c)], rbc)
        pltpu.sync_copy(ecc_c.at[pl.ds(ebase_c, win_c)], cbc)
        _stage_edges(rbc, cbc, idxc, valc, ebase_c, lo_c, lo_c + per_c, _LD_C)

        lo_d = wid * per_d
        pltpu.sync_copy(edd_r.at[pl.ds(lo_d, win_d)], rbd)
        pltpu.sync_copy(edd_c.at[pl.ds(lo_d, win_d)], cbd)
        _stage_edges(rbd, cbd, idxd, vald, lo_d, lo_d, lo_d + per_d, _LD_D)

        plsc.subcore_barrier()

        scopies = [pltpu.async_copy(valc.at[j], sh_cc.at[idxc.at[j]], sem,
                                    add=True) for j in range(nb_c)]
        scopies += [pltpu.async_copy(vald.at[j], sh_dd.at[idxd.at[j]], sem,
                                     add=True) for j in range(nb_d)]
        for cp in scopies:
            cp.wait()

        plsc.subcore_barrier()

        pltpu.sync_copy(sh_cc.at[pl.ds(sid * _CHK_C, _CHK_C)],
                        out_cc.at[cid, pl.ds(sid * _CHK_C, _CHK_C)])
        pltpu.sync_copy(sh_dd.at[pl.ds(sid * _CHK_D, _CHK_D)],
                        out_dd.at[cid, pl.ds(sid * _CHK_D, _CHK_D)])

    return k(rows_cc, cols_cc, rows_dd, cols_dd)


def _dot_nt(a, b):
    return lax.dot_general(a, b, (((1,), (1,)), ((), ())),
                           preferred_element_type=jnp.float32)


def _dot_nn(a, b):
    return lax.dot_general(a, b, (((1,), (0,)), ((), ())),
                           preferred_element_type=jnp.float32)


def _hist_t(edges_ref, cnt_ref, n, nb):
    e = edges_ref.shape[1]
    chunk = e // nb
    r_all = edges_ref[0:1, :]
    c_all = edges_ref[1:2, :]
    for i in range(nb):
        rows = r_all[:, i * chunk:(i + 1) * chunk]
        cols = c_all[:, i * chunk:(i + 1) * chunk]
        iota = lax.broadcasted_iota(jnp.int32, (n, chunk), 0)
        rt = jnp.where(iota == jnp.broadcast_to(rows, (n, chunk)),
                       1.0, 0.0).astype(jnp.bfloat16)
        ct = jnp.where(iota == jnp.broadcast_to(cols, (n, chunk)),
                       1.0, 0.0).astype(jnp.bfloat16)
        acc = _dot_nt(ct, rt)
        if i == 0:
            cnt_ref[...] = acc
        else:
            cnt_ref[...] += acc


def _gcn(st, xw, dinv, b):
    out = _dot_nn(st, xw * dinv) * dinv
    return jax.nn.relu(out + dinv * dinv * xw + b)


def _gat(cnt_t, mat_t, st, x1, wg, asrc, adst, we, ae, einv, bg):
    m = x1.shape[0]
    xs = _dot_nt(x1, wg)
    mean_ea = jnp.sum(st, axis=1, keepdims=True).sum(axis=0, keepdims=True) * einv
    present = cnt_t > 0.0
    acc = jnp.zeros((m, C), jnp.float32)
    for h in range(H):
        xs_h = xs[:, h * C:(h + 1) * C]
        asrc_h = asrc[h:h + 1, :]
        adst_h = adst[h:h + 1, :]
        coeff_h = jnp.sum(we[h:h + 1, :] * ae[h:h + 1, :], axis=1,
                          keepdims=True)
        a_src_col = jnp.sum(xs_h * asrc_h, axis=1, keepdims=True)
        a_dst_col = jnp.sum(xs_h * adst_h, axis=1, keepdims=True)
        a_src_row = _dot_nt(asrc_h, xs_h)
        alpha = a_dst_col + a_src_row + mat_t * coeff_h
        alpha = jnp.where(alpha > 0, alpha, 0.2 * alpha)
        aloop = a_src_col + a_dst_col + mean_ea * coeff_h
        aloop = jnp.where(aloop > 0, aloop, 0.2 * aloop)
        amax = jnp.max(jnp.where(present, alpha, -1e30), axis=1, keepdims=True)
        amax = jnp.maximum(amax, aloop)
        ex = cnt_t * jnp.exp(jnp.where(present, alpha - amax, -30.0))
        exl = jnp.exp(aloop - amax)
        den = jnp.sum(ex, axis=1, keepdims=True) + exl
        num = _dot_nn(ex, xs_h) + exl * xs_h
        acc = acc + num / (den + 1e-16)
    return jax.nn.relu(acc * (1.0 / H) + bg)


def _branch(cnt_t, mat, x, w1, b1, wg, asrc, adst, we2, ae, einv, bg, w2, b2):
    mat_t = mat.T
    st = cnt_t * mat_t
    dinv = lax.rsqrt(jnp.sum(st, axis=1, keepdims=True) + 1.0)
    x1 = _gcn(st, _dot_nt(x, w1), dinv, b1)
    xa = _gat(cnt_t, mat_t, st, x1, wg, asrc, adst, we2, ae, einv, bg)
    x2 = _gcn(st, _dot_nt(xa, w2), dinv, b2)
    return x1, x2


def _body(e_cc, e_dd,
          hc_ref, hd_ref, mat_c, mat_d, x_c, x_d,
          w1c, b1c, wgc, asrc_c, adst_c, we2c, aec, bgc, w2c, b2c,
          w1d, b1d, wgd, asrc_d, adst_d, we2d, aed, bgd, w2d, b2d,
          wcc, bcc, wcd, bcd,
          score_ref, cir_ref, dis_ref):
    hc = hc_ref[...]
    hd = hd_ref[...]
    cnt_c = (hc[0] + hc[1])[:N_CIR, :N_CIR]
    cnt_d = (hd[0] + hd[1])[:N_DIS, :N_DIS]
    x1, x2 = _branch(cnt_c, mat_c[...], x_c[...], w1c[...],
                     b1c[...][None, :],
                     wgc[...], asrc_c[...], adst_c[...], we2c[...], aec[...],
                     1.0 / e_cc, bgc[...][None, :], w2c[...],
                     b2c[...][None, :])
    y1, y2 = _branch(cnt_d, mat_d[...], x_d[...], w1d[...],
                     b1d[...][None, :],
                     wgd[...], asrc_d[...], adst_d[...], we2d[...], aed[...],
                     1.0 / e_dd, bgd[...][None, :], w2d[...],
                     b2d[...][None, :])
    cir = _dot_nt(jnp.concatenate([x1, x2], axis=1), wcc[...]) + bcc[...][None, :]
    dis = _dot_nt(jnp.concatenate([y1, y2], axis=1), wcd[...]) + bcd[...][None, :]
    cir_ref[...] = cir
    dis_ref[...] = dis
    score_ref[...] = _dot_nt(cir, dis)


def kernel(cc_matrix, cc_edges, dd_matrix, dd_edges, x_cir, x_dis,
           W1c, b1c, Wgc, asrc_c, adst_c, We_c, ae_c, bg_c, W2c, b2c,
           W1d, b1d, Wgd, asrc_d, adst_d, We_d, ae_d, bg_d, W2d, b2d,
           Wcnn_c, bcnn_c, Wcnn_d, bcnn_d):
    e_cc = cc_edges.shape[1]
    e_dd = dd_edges.shape[1]

    hist_cc, hist_dd = _sc_hist(cc_edges[0], cc_edges[1],
                                dd_edges[0], dd_edges[1])
    hist_cc = hist_cc.reshape(_NC, _PR_C, _LD_C)
    hist_dd = hist_dd.reshape(_NC, _PR_D, _LD_D)

    out_shapes = (
        jax.ShapeDtypeStruct((N_CIR, N_DIS), jnp.float32),
        jax.ShapeDtypeStruct((N_CIR, 2 * C), jnp.float32),
        jax.ShapeDtypeStruct((N_DIS, 2 * C), jnp.float32),
    )
    return pl.pallas_call(
        functools.partial(_body, float(e_cc), float(e_dd)),
        out_shape=out_shapes,
    )(hist_cc, hist_dd, cc_matrix, dd_matrix, x_cir, x_dis,
      W1c, b1c, Wgc, asrc_c, adst_c, We_c.reshape(8, 128), ae_c,
      bg_c, W2c, b2c,
      W1d, b1d, Wgd, asrc_d, adst_d, We_d.reshape(8, 128), ae_d,
      bg_d, W2d, b2d,
      Wcnn_c, bcnn_c, Wcnn_d, bcnn_d)


# speedup vs baseline: 1.1253x; 1.0028x over previous
import functools
import jax
import jax.numpy as jnp
from jax import lax
from jax.experimental import pallas as pl
from jax.experimental.pallas import tpu as pltpu
from jax.experimental.pallas import tpu_sc as plsc

N_CIR = 585
N_DIS = 88
H = 8
C = 128

_NC = 2
_NS = 16
_NW = _NC * _NS

_PR_C, _LD_C = 592, 640
_PR_D, _LD_D = 96, 128
_HW_C = _PR_C * _LD_C
_HW_D = _PR_D * _LD_D
_CHK_C = _HW_C // _NS
_CHK_D = _HW_D // _NS


def _stage_edges(rows_v, cols_v, idx_v, val_v, ebase, lo, hi, ld):
    n = rows_v.shape[0]
    nb, bw = idx_v.shape
    lanes = lax.iota(jnp.int32, 16)
    for i in range(n // 16):
        r = rows_v[pl.ds(i * 16, 16)]
        c = cols_v[pl.ds(i * 16, 16)]
        g = ebase + i * 16 + lanes
        ok = (g >= lo) & (g < hi)
        lin = c * ld + r
        j, k = (i * 16) // bw, (i * 16) % bw
        idx_v[j, pl.ds(k, 16)] = jnp.where(ok, lin, 0)
        val_v[j, pl.ds(k, 16)] = jnp.where(ok, 1.0, 0.0)
    for t in range(n, nb * bw, 16):
        j, k = t // bw, t % bw
        idx_v[j, pl.ds(k, 16)] = jnp.zeros((16,), jnp.int32)
        val_v[j, pl.ds(k, 16)] = jnp.zeros((16,), jnp.float32)


def _sc_hist(rows_cc, cols_cc, rows_dd, cols_dd):
    e_cc = rows_cc.shape[0]
    e_dd = rows_dd.shape[0]
    per_c = e_cc // _NW
    win_c = ((per_c + 7) // 8) * 8 + 8
    nb_c = (win_c + 127) // 128
    per_d = e_dd // _NW
    win_d = per_d
    nb_d = (win_d + 127) // 128

    mesh = plsc.VectorSubcoreMesh(core_axis_name="c", subcore_axis_name="s")

    @functools.partial(
        pl.kernel, mesh=mesh,
        out_type=(jax.ShapeDtypeStruct((_NC, _HW_C), jnp.float32),
                  jax.ShapeDtypeStruct((_NC, _HW_D), jnp.float32)),
        scratch_types=[
            pltpu.VMEM((win_c,), jnp.int32),
            pltpu.VMEM((win_c,), jnp.int32),
            pltpu.VMEM((nb_c, 128), jnp.int32),
            pltpu.VMEM((nb_c, 128), jnp.float32),
            pltpu.VMEM((win_d,), jnp.int32),
            pltpu.VMEM((win_d,), jnp.int32),
            pltpu.VMEM((nb_d, 128), jnp.int32),
            pltpu.VMEM((nb_d, 128), jnp.float32),
            pltpu.VMEM((_CHK_C // 10,), jnp.float32),
            pltpu.VMEM_SHARED((_HW_C,), jnp.float32),
            pltpu.VMEM_SHARED((_HW_D,), jnp.float32),
            pltpu.SemaphoreType.DMA,
        ],
    )
    def k(ecc_r, ecc_c, edd_r, edd_c, out_cc, out_dd, rbc, cbc, idxc, valc,
          rbd, cbd, idxd, vald, zbuf, sh_cc, sh_dd, sem):
        cid = lax.axis_index("c")
        sid = lax.axis_index("s")
        wid = sid * _NC + cid

        zc = _CHK_C // 10
        def zstep(i, _):
            zbuf[pl.ds(i * 16, 16)] = jnp.zeros((16,), jnp.float32)
            return 0
        lax.fori_loop(0, zc // 16, zstep, 0)
        zcopies = [pltpu.async_copy(
            zbuf, sh_cc.at[pl.ds(sid * _CHK_C + j * zc, zc)], sem)
            for j in range(10)]
        zcopies.append(pltpu.async_copy(
            zbuf.at[pl.ds(0, _CHK_D)],
            sh_dd.at[pl.ds(sid * _CHK_D, _CHK_D)], sem))
        for cp in zcopies:
            cp.wait()

        lo_c = wid * per_c
        ebase_c = jnp.minimum(lo_c // 8 * 8, e_cc - win_c)
        pltpu.sync_copy(ecc_r.at[pl.ds(ebase_c, win_c)], rbc)
        pltpu.sync_copy(ecc_c.at[pl.ds(ebase_c, win_c)], cbc)
        _stage_edges(rbc, cbc, idxc, valc, ebase_c, lo_c, lo_c + per_c, _LD_C)

        lo_d = wid * per_d
        pltpu.sync_copy(edd_r.at[pl.ds(lo_d, win_d)], rbd)
        pltpu.sync_copy(edd_c.at[pl.ds(lo_d, win_d)], cbd)
        _stage_edges(rbd, cbd, idxd, vald, lo_d, lo_d, lo_d + per_d, _LD_D)

        plsc.subcore_barrier()

        scopies = [pltpu.async_copy(valc.at[j], sh_cc.at[idxc.at[j]], sem,
                                    add=True) for j in range(nb_c)]
        scopies += [pltpu.async_copy(vald.at[j], sh_dd.at[idxd.at[j]], sem,
                                     add=True) for j in range(nb_d)]
        for cp in scopies:
            cp.wait()

        plsc.subcore_barrier()

        pltpu.sync_copy(sh_cc.at[pl.ds(sid * _CHK_C, _CHK_C)],
                        out_cc.at[cid, pl.ds(sid * _CHK_C, _CHK_C)])
        pltpu.sync_copy(sh_dd.at[pl.ds(sid * _CHK_D, _CHK_D)],
                        out_dd.at[cid, pl.ds(sid * _CHK_D, _CHK_D)])

    return k(rows_cc, cols_cc, rows_dd, cols_dd)


def _dot_nt(a, b):
    return lax.dot_general(a, b, (((1,), (1,)), ((), ())),
                           preferred_element_type=jnp.float32)


def _dot_nn(a, b):
    return lax.dot_general(a, b, (((1,), (0,)), ((), ())),
                           preferred_element_type=jnp.float32)


def _gcn(st, xw, dinv, b):
    out = _dot_nn(st, xw * dinv) * dinv
    return jax.nn.relu(out + dinv * dinv * xw + b)


def _gat(cnt_t, mat_t, st, x1, wg, asrc, adst, we, ae, einv, bg):
    m = x1.shape[0]
    xs = _dot_nt(x1, wg)
    mean_ea = jnp.sum(st, axis=1, keepdims=True).sum(axis=0, keepdims=True) * einv
    present = cnt_t > 0.0
    acc = jnp.zeros((m, C), jnp.float32)
    for h in range(H):
        xs_h = xs[:, h * C:(h + 1) * C]
        asrc_h = asrc[h:h + 1, :]
        adst_h = adst[h:h + 1, :]
        coeff_h = jnp.sum(we[h:h + 1, :] * ae[h:h + 1, :], axis=1,
                          keepdims=True)
        a_src_col = jnp.sum(xs_h * asrc_h, axis=1, keepdims=True)
        a_dst_col = jnp.sum(xs_h * adst_h, axis=1, keepdims=True)
        a_src_row = _dot_nt(asrc_h, xs_h)
        alpha = a_dst_col + a_src_row + mat_t * coeff_h
        alpha = jnp.where(alpha > 0, alpha, 0.2 * alpha)
        aloop = a_src_col + a_dst_col + mean_ea * coeff_h
        aloop = jnp.where(aloop > 0, aloop, 0.2 * aloop)
        amax = jnp.max(jnp.where(present, alpha, -1e30), axis=1, keepdims=True)
        amax = jnp.maximum(amax, aloop)
        ex = cnt_t * jnp.exp(jnp.where(present, alpha - amax, -30.0))
        exl = jnp.exp(aloop - amax)
        den = jnp.sum(ex, axis=1, keepdims=True) + exl
        num = _dot_nn(ex, xs_h) + exl * xs_h
        acc = acc + num / (den + 1e-16)
    return jax.nn.relu(acc * (1.0 / H) + bg)


def _branch(cnt_t, mat, x, w1, b1, wg, asrc, adst, we2, ae, einv, bg, w2, b2):
    mat_t = mat.T
    st = cnt_t * mat_t
    dinv = lax.rsqrt(jnp.sum(st, axis=1, keepdims=True) + 1.0)
    x1 = _gcn(st, _dot_nt(x, w1), dinv, b1)
    xa = _gat(cnt_t, mat_t, st, x1, wg, asrc, adst, we2, ae, einv, bg)
    x2 = _gcn(st, _dot_nt(xa, w2), dinv, b2)
    return x1, x2


def _body(e_cc, e_dd,
          hc_ref, hd_ref, mat_c, mat_d, x_c, x_d,
          w1c, b1c, wgc, asrc_c, adst_c, we2c, aec, bgc, w2c, b2c,
          w1d, b1d, wgd, asrc_d, adst_d, we2d, aed, bgd, w2d, b2d,
          wcc, bcc, wcd, bcd,
          score_ref, cir_ref, dis_ref):
    hc = hc_ref[...]
    hd = hd_ref[...]
    cnt_c = (hc[0] + hc[1])[:N_CIR, :N_CIR]
    cnt_d = (hd[0] + hd[1])[:N_DIS, :N_DIS]
    x1, x2 = _branch(cnt_c, mat_c[...], x_c[...], w1c[...],
                     b1c[...][None, :],
                     wgc[...], asrc_c[...], adst_c[...], we2c[...], aec[...],
                     1.0 / e_cc, bgc[...][None, :], w2c[...],
                     b2c[...][None, :])
    y1, y2 = _branch(cnt_d, mat_d[...], x_d[...], w1d[...],
                     b1d[...][None, :],
                     wgd[...], asrc_d[...], adst_d[...], we2d[...], aed[...],
                     1.0 / e_dd, bgd[...][None, :], w2d[...],
                     b2d[...][None, :])
    cir = _dot_nt(jnp.concatenate([x1, x2], axis=1), wcc[...]) + bcc[...][None, :]
    dis = _dot_nt(jnp.concatenate([y1, y2], axis=1), wcd[...]) + bcd[...][None, :]
    cir_ref[...] = cir
    dis_ref[...] = dis
    score_ref[...] = _dot_nt(cir, dis)


def kernel(cc_matrix, cc_edges, dd_matrix, dd_edges, x_cir, x_dis,
           W1c, b1c, Wgc, asrc_c, adst_c, We_c, ae_c, bg_c, W2c, b2c,
           W1d, b1d, Wgd, asrc_d, adst_d, We_d, ae_d, bg_d, W2d, b2d,
           Wcnn_c, bcnn_c, Wcnn_d, bcnn_d):
    e_cc = cc_edges.shape[1]
    e_dd = dd_edges.shape[1]

    hist_cc, hist_dd = _sc_hist(cc_edges[0], cc_edges[1],
                                dd_edges[0], dd_edges[1])
    hist_cc = hist_cc.reshape(_NC, _PR_C, _LD_C)
    hist_dd = hist_dd.reshape(_NC, _PR_D, _LD_D)

    out_shapes = (
        jax.ShapeDtypeStruct((N_CIR, N_DIS), jnp.float32),
        jax.ShapeDtypeStruct((N_CIR, 2 * C), jnp.float32),
        jax.ShapeDtypeStruct((N_DIS, 2 * C), jnp.float32),
    )
    return pl.pallas_call(
        functools.partial(_body, float(e_cc), float(e_dd)),
        out_shape=out_shapes,
    )(hist_cc, hist_dd, cc_matrix, dd_matrix, x_cir, x_dis,
      W1c, b1c, Wgc, asrc_c, adst_c, We_c.reshape(8, 128), ae_c,
      bg_c, W2c, b2c,
      W1d, b1d, Wgd, asrc_d, adst_d, We_d.reshape(8, 128), ae_d,
      bg_d, W2d, b2d,
      Wcnn_c, bcnn_c, Wcnn_d, bcnn_d)
